# Initial kernel scaffold; baseline (speedup 1.0000x reference)
import jax
import jax.numpy as jnp
from jax.experimental import pallas as pl


def kernel(var_node_features, con_node_features, edge_features_var, edge_features_con, edge_index_var, edge_index_con, params):
    raise NotImplementedError("write your pallas kernel here")


# speedup vs baseline: 2.0884x; 2.0884x over previous
import functools

import jax
import jax.numpy as jnp
from jax import lax
from jax.experimental import pallas as pl
from jax.experimental.pallas import tpu as pltpu
from jax.experimental.pallas import tpu_sc as plsc

_H = 64
_W = 128
_L = 2
_NV = 25000
_NC = 25000
_E = 800000

_EB = 8000
_CH = 128
_SPLIT = 12512
_ACC = 12520
_NOUT = 25008


def _sc_info():
    info = plsc.get_sparse_core_info()
    return info.num_cores, info.num_subcores


def _sc_gather(table, idx):
    NCOR, NSUB = _sc_info()
    NW = NCOR * NSUB
    B = idx.shape[0]
    n_chunks = B // _CH
    per_w = -(-n_chunks // NW)
    mesh = plsc.VectorSubcoreMesh(core_axis_name="c", subcore_axis_name="s")

    @functools.partial(
        pl.kernel, mesh=mesh,
        out_type=jax.ShapeDtypeStruct((B, _W), jnp.float32),
        scratch_types=[
            pltpu.VMEM((_CH,), jnp.int32),
            pltpu.VMEM((_CH, _W), jnp.float32),
            pltpu.SemaphoreType.DMA,
        ],
    )
    def k(table_hbm, idx_hbm, out_hbm, idx_v, rows_v, sem):
        wid = lax.axis_index("s") * NCOR + lax.axis_index("c")

        def body(i, carry):
            c = i * NW + wid

            @pl.when(c < n_chunks)
            def _():
                off = c * _CH
                pltpu.sync_copy(idx_hbm.at[pl.ds(off, _CH)], idx_v)
                pltpu.async_copy(table_hbm.at[idx_v], rows_v, sem).wait()
                pltpu.sync_copy(rows_v, out_hbm.at[pl.ds(off, _CH)])

            return carry

        lax.fori_loop(0, per_w, body, 0)

    return k(table, idx)


def _sc_scatter_add(msg, idx):
    NCOR, NSUB = _sc_info()
    B = idx.shape[0]
    n_chunks = B // _CH
    per_w = -(-n_chunks // NSUB)
    zeros = jnp.zeros((_ACC, _W), jnp.float32)
    mesh = plsc.VectorSubcoreMesh(core_axis_name="c", subcore_axis_name="s")

    @functools.partial(
        pl.kernel, mesh=mesh,
        out_type=jax.ShapeDtypeStruct((_NOUT, _W), jnp.float32),
        scratch_types=[
            pltpu.VMEM((_CH,), jnp.int32),
            pltpu.VMEM((_CH,), jnp.int32),
            pltpu.VMEM((_CH, _W), jnp.float32),
            pltpu.VMEM_SHARED((_ACC, _W), jnp.float32),
            pltpu.SemaphoreType.DMA,
        ],
    )
    def k(zeros_hbm, msg_hbm, idx_hbm, out_hbm, idx_v, loc_v, rows_v,
          acc_sh, sem):
        cid = lax.axis_index("c")
        sid = lax.axis_index("s")
        base = cid * _SPLIT
        bound = jnp.where(cid == 0, _SPLIT, _NOUT - _SPLIT)

        @pl.when(sid == 0)
        def _():
            pltpu.sync_copy(zeros_hbm, acc_sh)

        plsc.subcore_barrier()

        def body(i, carry):
            c = i * NSUB + sid

            @pl.when(c < n_chunks)
            def _():
                off = c * _CH
                pltpu.sync_copy(idx_hbm.at[pl.ds(off, _CH)], idx_v)
                pltpu.sync_copy(msg_hbm.at[pl.ds(off, _CH)], rows_v)
                for r in range(_CH // 16):
                    v = idx_v[pl.ds(r * 16, 16)] - base
                    oob = (v < 0) | (v >= bound)
                    loc_v[pl.ds(r * 16, 16)] = jnp.where(
                        oob, _ACC - 8, v)
                pltpu.sync_copy(rows_v, acc_sh.at[loc_v], add=True)

            return carry

        lax.fori_loop(0, per_w, body, 0)
        plsc.subcore_barrier()

        @pl.when((sid == 0) & (cid == 0))
        def _():
            pltpu.sync_copy(acc_sh.at[pl.ds(0, _SPLIT)],
                            out_hbm.at[pl.ds(0, _SPLIT)])

        @pl.when((sid == 0) & (cid == 1))
        def _():
            pltpu.sync_copy(acc_sh.at[pl.ds(0, _NOUT - _SPLIT)],
                            out_hbm.at[pl.ds(_SPLIT, _NOUT - _SPLIT)])

    return k(zeros, msg, idx)


def _enc_body(x_ref, w1_ref, b1_ref, w2_ref, b2_ref, o_ref):
    h = jnp.maximum(
        jnp.dot(x_ref[...], w1_ref[...], preferred_element_type=jnp.float32)
        + b1_ref[...], 0.0)
    v = (jnp.dot(h, w2_ref[...], preferred_element_type=jnp.float32)
         + b2_ref[...])
    o_ref[...] = jnp.concatenate(
        [v, jnp.zeros((v.shape[0], _W - _H), jnp.float32)], axis=1)


def _encode(x, p):
    return pl.pallas_call(
        _enc_body,
        out_shape=jax.ShapeDtypeStruct((x.shape[0], _W), jnp.float32),
    )(x, p['W1'], p['b1'].reshape(1, _H), p['W2'], p['b2'].reshape(1, _H))


def _edge_mlp(a, we1, be1, we2, be2):
    e1 = jnp.maximum(a * we1 + be1, 0.0)
    return jnp.maximum(
        jnp.dot(e1, we2, preferred_element_type=jnp.float32) + be2, 0.0)


def _stats_body(a_ref, we1_ref, be1_ref, we2_ref, be2_ref, o_ref):
    i = pl.program_id(0)
    e2 = _edge_mlp(a_ref[...], we1_ref[...], be1_ref[...], we2_ref[...],
                   be2_ref[...])
    s = jnp.sum(e2, axis=0, keepdims=True)
    sq = jnp.sum(e2 * e2, axis=0, keepdims=True)
    blk = jnp.concatenate([s, sq, jnp.zeros((6, _H), jnp.float32)], axis=0)

    @pl.when(i == 0)
    def _():
        o_ref[...] = blk

    @pl.when(i > 0)
    def _():
        o_ref[...] += blk


def _edge_stats(attr, p):
    return pl.pallas_call(
        _stats_body,
        grid=(_E // _EB,),
        in_specs=[
            pl.BlockSpec((_EB, 1), lambda i: (i, 0)),
            pl.BlockSpec((1, _H), lambda i: (0, 0)),
            pl.BlockSpec((1, _H), lambda i: (0, 0)),
            pl.BlockSpec((_H, _H), lambda i: (0, 0)),
            pl.BlockSpec((1, _H), lambda i: (0, 0)),
        ],
        out_specs=pl.BlockSpec((8, _H), lambda i: (0, 0)),
        out_shape=jax.ShapeDtypeStruct((8, _H), jnp.float32),
    )(attr, p['We1'], p['be1'].reshape(1, _H), p['We2'],
      p['be2'].reshape(1, _H))


def _msg_body(a_ref, g_ref, we1_ref, be1_ref, we2_ref, be2_ref, st_ref,
              gam_ref, bet_ref, o_ref):
    e2 = _edge_mlp(a_ref[...], we1_ref[...], be1_ref[...], we2_ref[...],
                   be2_ref[...])
    n = jnp.float32(_E)
    mu = st_ref[0:1, :] / n
    var = st_ref[1:2, :] / n - mu * mu
    scale = gam_ref[...] * lax.rsqrt(var + 1e-5)
    shift = bet_ref[...] - mu * scale
    m = jnp.maximum(g_ref[:, :_H] + e2 * scale + shift, 0.0)
    pad = jnp.concatenate(
        [jnp.ones((m.shape[0], 1), jnp.float32),
         jnp.zeros((m.shape[0], _W - _H - 1), jnp.float32)], axis=1)
    o_ref[...] = jnp.concatenate([m, pad], axis=1)


def _messages(attr, gathered, stats, p):
    return pl.pallas_call(
        _msg_body,
        grid=(_E // _EB,),
        in_specs=[
            pl.BlockSpec((_EB, 1), lambda i: (i, 0)),
            pl.BlockSpec((_EB, _W), lambda i: (i, 0)),
            pl.BlockSpec((1, _H), lambda i: (0, 0)),
            pl.BlockSpec((1, _H), lambda i: (0, 0)),
            pl.BlockSpec((_H, _H), lambda i: (0, 0)),
            pl.BlockSpec((1, _H), lambda i: (0, 0)),
            pl.BlockSpec((8, _H), lambda i: (0, 0)),
            pl.BlockSpec((1, _H), lambda i: (0, 0)),
            pl.BlockSpec((1, _H), lambda i: (0, 0)),
        ],
        out_specs=pl.BlockSpec((_EB, _W), lambda i: (i, 0)),
        out_shape=jax.ShapeDtypeStruct((_E, _W), jnp.float32),
    )(attr, gathered, p['We1'], p['be1'].reshape(1, _H), p['We2'],
      p['be2'].reshape(1, _H), stats, p['g'].reshape(1, _H),
      p['b'].reshape(1, _H))


def _out_body(agg_ref, t_ref, wl_ref, bl_ref, wr_ref, o_ref):
    agg = agg_ref[:, :_H]
    cnt = agg_ref[:, _H:_H + 1]
    mean = agg / jnp.maximum(cnt, 1.0)
    out = (jnp.dot(mean, wl_ref[...], preferred_element_type=jnp.float32)
           + bl_ref[...]
           + jnp.dot(t_ref[:, :_H], wr_ref[...],
                     preferred_element_type=jnp.float32))
    nrm = jnp.maximum(
        jnp.sqrt(jnp.sum(out * out, axis=-1, keepdims=True)), 1e-12)
    v = jnp.maximum(out / nrm, 0.0)
    o_ref[...] = jnp.concatenate(
        [v, jnp.zeros((v.shape[0], _W - _H), jnp.float32)], axis=1)


def _node_update(agg, target, p):
    nb = 5000
    n = target.shape[0]
    return pl.pallas_call(
        _out_body,
        grid=(n // nb,),
        in_specs=[
            pl.BlockSpec((nb, _W), lambda i: (i, 0)),
            pl.BlockSpec((nb, _W), lambda i: (i, 0)),
            pl.BlockSpec((_H, _H), lambda i: (0, 0)),
            pl.BlockSpec((1, _H), lambda i: (0, 0)),
            pl.BlockSpec((_H, _H), lambda i: (0, 0)),
        ],
        out_specs=pl.BlockSpec((nb, _W), lambda i: (i, 0)),
        out_shape=jax.ShapeDtypeStruct((n, _W), jnp.float32),
    )(agg, target, p['Wl'], p['bl'].reshape(1, _H), p['Wr'])


def _final_body(x_ref, w1_ref, b1_ref, w2_ref, b2_ref, w3_ref, b3_ref,
                w4_ref, b4_ref, o_ref):
    h = jnp.maximum(
        jnp.dot(x_ref[...], w1_ref[...], preferred_element_type=jnp.float32)
        + b1_ref[...], 0.0)
    h = jnp.maximum(
        jnp.dot(h, w2_ref[...], preferred_element_type=jnp.float32)
        + b2_ref[...], 0.0)
    h = jnp.maximum(
        jnp.dot(h, w3_ref[...], preferred_element_type=jnp.float32)
        + b3_ref[...], 0.0)
    y = (jnp.dot(h, w4_ref[...], preferred_element_type=jnp.float32)
         + b4_ref[...])
    m = jnp.max(y, axis=-1, keepdims=True)
    o_ref[...] = y - (
        m + jnp.log(jnp.sum(jnp.exp(y - m), axis=-1, keepdims=True)))


def _final_mlp(x, p):
    nb = 5000
    d_in = x.shape[1]
    return pl.pallas_call(
        _final_body,
        grid=(_NV // nb,),
        in_specs=[
            pl.BlockSpec((nb, d_in), lambda i: (i, 0)),
            pl.BlockSpec((d_in, _H), lambda i: (0, 0)),
            pl.BlockSpec((1, _H), lambda i: (0, 0)),
            pl.BlockSpec((_H, _H), lambda i: (0, 0)),
            pl.BlockSpec((1, _H), lambda i: (0, 0)),
            pl.BlockSpec((_H, _H), lambda i: (0, 0)),
            pl.BlockSpec((1, _H), lambda i: (0, 0)),
            pl.BlockSpec((_H, 2), lambda i: (0, 0)),
            pl.BlockSpec((1, 2), lambda i: (0, 0)),
        ],
        out_specs=pl.BlockSpec((nb, 2), lambda i: (i, 0)),
        out_shape=jax.ShapeDtypeStruct((_NV, 2), jnp.float32),
    )(x, p['W1'], p['b1'].reshape(1, _H), p['W2'], p['b2'].reshape(1, _H),
      p['W3'], p['b3'].reshape(1, _H), p['W4'], p['b4'].reshape(1, 2))


def _bipartite(p, source, target, src, dst, attr):
    stats = _edge_stats(attr, p)
    gathered = _sc_gather(source, src)
    msg = _messages(attr, gathered, stats, p)
    agg = _sc_scatter_add(msg, dst)
    return _node_update(agg[:target.shape[0], :], target, p)


def kernel(var_node_features, con_node_features, edge_features_var,
           edge_features_con, edge_index_var, edge_index_con, params):
    p = params
    src_v = edge_index_var[0].astype(jnp.int32)
    dst_c = edge_index_var[1].astype(jnp.int32)
    src_c = edge_index_con[0].astype(jnp.int32)
    dst_v = edge_index_con[1].astype(jnp.int32)

    xv = [_encode(var_node_features, p['enc_v'])]
    xc = [_encode(con_node_features, p['enc_c'])]
    for i in range(_L):
        xc.append(_bipartite(p['lv'][i], xv[-1], xc[-1], src_v, dst_c,
                             edge_features_var))
        xv.append(_bipartite(p['lc'][i], xc[-1], xv[-1], src_c, dst_v,
                             edge_features_con))

    x = jnp.concatenate([f[:, :_H] for f in xv], axis=-1)
    return _final_mlp(x, p)
